# baseline (speedup 1.0000x reference)
import functools

import jax
import jax.numpy as jnp
from jax import lax
from jax.experimental import pallas as pl
from jax.experimental.pallas import tpu as pltpu
from jax.experimental.pallas import tpu_sc as plsc

_BATCH = 16384
_DIM = 32
_EPS = 1e-8

_info = plsc.get_sparse_core_info()
_NC, _NS, _L = _info.num_cores, _info.num_subcores, _info.num_lanes
_NW = _NC * _NS
_PAIRS_PER_W = _BATCH // _NW
_ROWS_PER_W = 2 * _PAIRS_PER_W
_CHUNK = 128
_NCHUNKS = _ROWS_PER_W // _CHUNK
_GROUPS = _PAIRS_PER_W // _L


def _rsqrt(s):
    i = lax.bitcast_convert_type(s, jnp.int32)
    y = lax.bitcast_convert_type(0x5F3759DF - (i >> 1), jnp.float32)
    for _ in range(3):
        y = y * (1.5 - 0.5 * s * y * y)
    return y


def _sc_body(idx_hbm, table_hbm, out_hbm, idx_v, rows_v, out_v, sem):
    wid = lax.axis_index("s") * _NC + lax.axis_index("c")

    pltpu.sync_copy(idx_hbm.at[wid], idx_v)

    copies = []
    for j in range(_NCHUNKS):
        copies.append(
            pltpu.make_async_copy(
                table_hbm.at[idx_v.at[j]],
                rows_v.at[pl.ds(j * _CHUNK, _CHUNK)],
                sem,
            )
        )
        copies[-1].start()
    for c in copies:
        c.wait()

    iota = lax.iota(jnp.int32, _L)

    def group(g, _):
        row1 = g * (2 * _L) + 2 * iota
        row2 = row1 + 1
        dot = jnp.zeros((_L,), jnp.float32)
        s1 = jnp.zeros((_L,), jnp.float32)
        s2 = jnp.zeros((_L,), jnp.float32)
        for d in range(_DIM):
            dsplat = jnp.full((_L,), d, jnp.int32)
            v1 = plsc.load_gather(rows_v, [row1, dsplat])
            v2 = plsc.load_gather(rows_v, [row2, dsplat])
            dot = dot + v1 * v2
            s1 = s1 + v1 * v1
            s2 = s2 + v2 * v2
        s = jnp.maximum(s1 * s2, jnp.float32(_EPS * _EPS))
        out_v[pl.ds(g * _L, _L)] = dot * _rsqrt(s)
        return _

    lax.fori_loop(0, _GROUPS, group, None)

    pltpu.sync_copy(out_v, out_hbm.at[pl.ds(wid * _PAIRS_PER_W, _PAIRS_PER_W)])


@functools.partial(
    pl.kernel,
    out_type=jax.ShapeDtypeStruct((_BATCH,), jnp.float32),
    mesh=plsc.VectorSubcoreMesh(core_axis_name="c", subcore_axis_name="s"),
    compiler_params=pltpu.CompilerParams(
        needs_layout_passes=False, use_tc_tiling_on_sc=False
    ),
    scratch_types=[
        pltpu.VMEM((_NCHUNKS, _CHUNK), jnp.int32),
        pltpu.VMEM((_ROWS_PER_W, _DIM), jnp.float32),
        pltpu.VMEM((_PAIRS_PER_W,), jnp.float32),
        pltpu.SemaphoreType.DMA,
    ],
)
def _sc_kernel(idx_hbm, table_hbm, out_hbm, idx_v, rows_v, out_v, sem):
    _sc_body(idx_hbm, table_hbm, out_hbm, idx_v, rows_v, out_v, sem)


def kernel(item_pairs, table):
    idx = item_pairs.reshape(_NW, _NCHUNKS, _CHUNK)
    return _sc_kernel(idx, table)


# speedup vs baseline: 1.6775x; 1.6775x over previous
import functools

import jax
import jax.numpy as jnp
from jax import lax
from jax.experimental import pallas as pl
from jax.experimental.pallas import tpu as pltpu
from jax.experimental.pallas import tpu_sc as plsc

_BATCH = 16384
_DIM = 32
_EPS = 1e-8

_info = plsc.get_sparse_core_info()
_NC, _NS, _L = _info.num_cores, _info.num_subcores, _info.num_lanes
_NW = _NC * _NS
_PAIRS_PER_W = _BATCH // _NW
_ROWS_PER_W = 2 * _PAIRS_PER_W
_GROUPS = _PAIRS_PER_W // _L


def _rsqrt(s):
    i = lax.bitcast_convert_type(s, jnp.int32)
    y = lax.bitcast_convert_type(0x5F3759DF - (i >> 1), jnp.float32)
    for _ in range(3):
        y = y * (1.5 - 0.5 * s * y * y)
    return y


def _sc_body(idx_hbm, table_hbm, out_hbm, idx_s, rows_v, out_v, sem):
    wid = lax.axis_index("s") * _NC + lax.axis_index("c")

    pltpu.sync_copy(idx_hbm.at[wid], idx_s)

    def fire(c, _):
        vec = idx_s[c >> 3, pl.ds((c & 7) * _L, _L)]
        for j in range(_L):
            q = c * _L + j
            i = vec[j]
            pltpu.make_async_copy(
                table_hbm.at[i, :],
                rows_v.at[q >> 2, pl.ds((q & 3) * _DIM, _DIM)],
                sem,
            ).start()
        return _

    lax.fori_loop(0, _ROWS_PER_W // _L, fire, None)

    def drain(j, _):
        pltpu.make_async_copy(
            table_hbm.at[0, :],
            rows_v.at[0, pl.ds(0, _DIM)],
            sem,
        ).wait()
        return _

    lax.fori_loop(0, _ROWS_PER_W, drain, None)

    iota = lax.iota(jnp.int32, _L)

    def group(g, _):
        base1 = g * (_L * _DIM) + _DIM * iota
        base2 = base1 + _PAIRS_PER_W * _DIM
        dot = jnp.zeros((_L,), jnp.float32)
        s1 = jnp.zeros((_L,), jnp.float32)
        s2 = jnp.zeros((_L,), jnp.float32)
        for t in range(_DIM):
            d = (iota + t) & (_DIM - 1)
            f1 = base1 + d
            f2 = base2 + d
            v1 = plsc.load_gather(rows_v, [f1 >> 7, f1 & 127])
            v2 = plsc.load_gather(rows_v, [f2 >> 7, f2 & 127])
            dot = dot + v1 * v2
            s1 = s1 + v1 * v1
            s2 = s2 + v2 * v2
        s = jnp.maximum(s1 * s2, jnp.float32(_EPS * _EPS))
        out_v[pl.ds(g * _L, _L)] = dot * _rsqrt(s)
        return _

    lax.fori_loop(0, _GROUPS, group, None)

    pltpu.sync_copy(out_v, out_hbm.at[pl.ds(wid * _PAIRS_PER_W, _PAIRS_PER_W)])


@functools.partial(
    pl.kernel,
    out_type=jax.ShapeDtypeStruct((_BATCH,), jnp.float32),
    mesh=plsc.VectorSubcoreMesh(core_axis_name="c", subcore_axis_name="s"),
    compiler_params=pltpu.CompilerParams(
        needs_layout_passes=False, use_tc_tiling_on_sc=True
    ),
    scratch_types=[
        pltpu.VMEM((8, 128), jnp.int32),
        pltpu.VMEM((_ROWS_PER_W * _DIM // 128, 128), jnp.float32),
        pltpu.VMEM((_PAIRS_PER_W,), jnp.float32),
        pltpu.SemaphoreType.DMA,
    ],
)
def _sc_kernel(idx_hbm, table_hbm, out_hbm, idx_s, rows_v, out_v, sem):
    _sc_body(idx_hbm, table_hbm, out_hbm, idx_s, rows_v, out_v, sem)


def kernel(item_pairs, table):
    idx = (
        item_pairs.T.reshape(2, _NW, 4, 128)
        .swapaxes(0, 1)
        .reshape(_NW, 8, 128)
    )
    return _sc_kernel(idx, table)


# speedup vs baseline: 1.6873x; 1.0059x over previous
import functools

import jax
import jax.numpy as jnp
from jax import lax
from jax.experimental import pallas as pl
from jax.experimental.pallas import tpu as pltpu
from jax.experimental.pallas import tpu_sc as plsc

_BATCH = 16384
_DIM = 32
_EPS = 1e-8

_info = plsc.get_sparse_core_info()
_NC, _NS, _L = _info.num_cores, _info.num_subcores, _info.num_lanes
_NW = _NC * _NS
_PAIRS_PER_W = _BATCH // _NW
_ROWS_PER_W = 2 * _PAIRS_PER_W
_GROUPS = _PAIRS_PER_W // _L


def _rsqrt(s):
    i = lax.bitcast_convert_type(s, jnp.int32)
    y = lax.bitcast_convert_type(0x5F3759DF - (i >> 1), jnp.float32)
    for _ in range(3):
        y = y * (1.5 - 0.5 * s * y * y)
    return y


def _sc_body(idx_hbm, table_hbm, out_hbm, idx_s, rows_v, out_v, sem, sem2):
    wid = lax.axis_index("s") * _NC + lax.axis_index("c")

    pltpu.sync_copy(idx_hbm.at[wid], idx_s)

    def fire(c, _):
        vec = idx_s[c >> 3, pl.ds((c & 7) * _L, _L)]
        for j in range(_L):
            q = c * _L + j
            i = vec[j]
            pltpu.make_async_copy(
                table_hbm.at[i, :],
                rows_v.at[q >> 2, pl.ds((q & 3) * _DIM, _DIM)],
                sem if j % 2 == 0 else sem2,
            ).start()
        return _

    lax.fori_loop(0, _ROWS_PER_W // _L, fire, None)

    def drain(j, _):
        pltpu.make_async_copy(
            table_hbm.at[0, :],
            rows_v.at[0, pl.ds(0, _DIM)],
            sem,
        ).wait()
        pltpu.make_async_copy(
            table_hbm.at[0, :],
            rows_v.at[0, pl.ds(0, _DIM)],
            sem2,
        ).wait()
        return _

    lax.fori_loop(0, _ROWS_PER_W // 2, drain, None)

    iota = lax.iota(jnp.int32, _L)

    def group(g, _):
        base1 = g * (_L * _DIM) + _DIM * iota
        base2 = base1 + _PAIRS_PER_W * _DIM
        dot = jnp.zeros((_L,), jnp.float32)
        s1 = jnp.zeros((_L,), jnp.float32)
        s2 = jnp.zeros((_L,), jnp.float32)
        for t in range(_DIM):
            d = (iota + t) & (_DIM - 1)
            f1 = base1 + d
            f2 = base2 + d
            v1 = plsc.load_gather(rows_v, [f1 >> 7, f1 & 127])
            v2 = plsc.load_gather(rows_v, [f2 >> 7, f2 & 127])
            dot = dot + v1 * v2
            s1 = s1 + v1 * v1
            s2 = s2 + v2 * v2
        s = jnp.maximum(s1 * s2, jnp.float32(_EPS * _EPS))
        out_v[pl.ds(g * _L, _L)] = dot * _rsqrt(s)
        return _

    lax.fori_loop(0, _GROUPS, group, None)

    pltpu.sync_copy(out_v, out_hbm.at[pl.ds(wid * _PAIRS_PER_W, _PAIRS_PER_W)])


@functools.partial(
    pl.kernel,
    out_type=jax.ShapeDtypeStruct((_BATCH,), jnp.float32),
    mesh=plsc.VectorSubcoreMesh(core_axis_name="c", subcore_axis_name="s"),
    compiler_params=pltpu.CompilerParams(
        needs_layout_passes=False, use_tc_tiling_on_sc=True
    ),
    scratch_types=[
        pltpu.VMEM((8, 128), jnp.int32),
        pltpu.VMEM((_ROWS_PER_W * _DIM // 128, 128), jnp.float32),
        pltpu.VMEM((_PAIRS_PER_W,), jnp.float32),
        pltpu.SemaphoreType.DMA,
        pltpu.SemaphoreType.DMA,
    ],
)
def _sc_kernel(idx_hbm, table_hbm, out_hbm, idx_s, rows_v, out_v, sem, sem2):
    _sc_body(idx_hbm, table_hbm, out_hbm, idx_s, rows_v, out_v, sem, sem2)


def kernel(item_pairs, table):
    idx = (
        item_pairs.T.reshape(2, _NW, 4, 128)
        .swapaxes(0, 1)
        .reshape(_NW, 8, 128)
    )
    return _sc_kernel(idx, table)
